# baseline (speedup 1.0000x reference)
import jax
import jax.numpy as jnp
from jax import lax
from jax.experimental import pallas as pl
from jax.experimental.pallas import tpu as pltpu
from jax.experimental.pallas import tpu_sc as plsc

BATCH = 16384
EMBED = 64
NC = 2
NS = 16
LANES = 16
NW = NC * NS
B_PER_W = BATCH // NW
CHUNK = 128
N_CHUNKS = B_PER_W // CHUNK
GROUPS = CHUNK // LANES


def _body(users_hbm, items_hbm, ut_hbm, it_hbm, out_hbm,
          uidx_v, iidx_v, urows_v, irows_v, out_v, usem, isem):
    wid = lax.axis_index("s") * NC + lax.axis_index("c")
    base = wid * B_PER_W

    for ch in range(N_CHUNKS):
        cbase = base + ch * CHUNK
        pltpu.sync_copy(users_hbm.at[pl.ds(cbase, CHUNK)], uidx_v)
        pltpu.sync_copy(items_hbm.at[pl.ds(cbase, CHUNK)], iidx_v)
        cu = pltpu.async_copy(ut_hbm.at[uidx_v], urows_v, usem)
        ci = pltpu.async_copy(it_hbm.at[iidx_v], irows_v, isem)
        cu.wait()
        ci.wait()

        lane_iota = lax.iota(jnp.int32, LANES)

        def group_body(g, _):
            acc = jnp.zeros((LANES,), jnp.float32)
            for j in range(LANES):
                r = g * LANES + j
                p = jnp.zeros((LANES,), jnp.float32)
                for c in range(EMBED // LANES):
                    u = urows_v[r, pl.ds(c * LANES, LANES)]
                    it = irows_v[r, pl.ds(c * LANES, LANES)]
                    p = p + u * it
                s = jnp.sum(p)
                acc = jnp.where(lane_iota == j, s, acc)
            res = 1.0 / (1.0 + jnp.exp(-acc))
            out_v[pl.ds(ch * CHUNK + g * LANES, LANES)] = res
            return 0

        lax.fori_loop(0, GROUPS, group_body, 0)

    pltpu.sync_copy(out_v, out_hbm.at[pl.ds(base, B_PER_W)])


@jax.jit
def kernel(users, items, user_table, item_table):
    mesh = plsc.VectorSubcoreMesh(core_axis_name="c", subcore_axis_name="s")
    k = pl.kernel(
        _body,
        out_type=jax.ShapeDtypeStruct((BATCH,), jnp.float32),
        mesh=mesh,
        compiler_params=pltpu.CompilerParams(
            use_tc_tiling_on_sc=False, needs_layout_passes=False),
        scratch_types=[
            pltpu.VMEM((CHUNK,), jnp.int32),
            pltpu.VMEM((CHUNK,), jnp.int32),
            pltpu.VMEM((CHUNK, EMBED), jnp.float32),
            pltpu.VMEM((CHUNK, EMBED), jnp.float32),
            pltpu.VMEM((B_PER_W,), jnp.float32),
            pltpu.SemaphoreType.DMA,
            pltpu.SemaphoreType.DMA,
        ],
    )
    return k(users, items, user_table, item_table)


# speedup vs baseline: 1.5433x; 1.5433x over previous
import jax
import jax.numpy as jnp
from jax import lax
from jax.experimental import pallas as pl
from jax.experimental.pallas import tpu as pltpu
from jax.experimental.pallas import tpu_sc as plsc

BATCH = 16384
EMBED = 64
NC = 2
NS = 16
LANES = 16
NW = NC * NS
B_PER_W = BATCH // NW
CHUNK = 32
N_CHUNKS = B_PER_W // CHUNK
GROUPS = CHUNK // LANES


def _body(users_hbm, items_hbm, ut_hbm, it_hbm, out_hbm,
          uidx_v, iidx_v, ubuf_v, ibuf_v, out_v, sem):
    wid = lax.axis_index("s") * NC + lax.axis_index("c")
    base = wid * B_PER_W
    lane = lax.iota(jnp.int32, LANES)

    def chunk_body(ch, _):
        cbase = base + ch * CHUNK
        pltpu.sync_copy(users_hbm.at[pl.ds(cbase, CHUNK)], uidx_v)
        pltpu.sync_copy(items_hbm.at[pl.ds(cbase, CHUNK)], iidx_v)
        copies = []
        for g in range(GROUPS):
            uvec = uidx_v[pl.ds(g * LANES, LANES)]
            ivec = iidx_v[pl.ds(g * LANES, LANES)]
            for j in range(LANES):
                ru = jnp.sum(jnp.where(lane == j, uvec, 0))
                ri = jnp.sum(jnp.where(lane == j, ivec, 0))
                r = g * LANES + j
                copies.append(
                    pltpu.async_copy(ut_hbm.at[ru], ubuf_v.at[r], sem))
                copies.append(
                    pltpu.async_copy(it_hbm.at[ri], ibuf_v.at[r], sem))
        for c in copies:
            c.wait()
        for g in range(GROUPS):
            acc = jnp.zeros((LANES,), jnp.float32)
            for j in range(LANES):
                r = g * LANES + j
                p = jnp.zeros((LANES,), jnp.float32)
                for c in range(EMBED // LANES):
                    u = ubuf_v[r, pl.ds(c * LANES, LANES)]
                    it = ibuf_v[r, pl.ds(c * LANES, LANES)]
                    p = p + u * it
                s = jnp.sum(p)
                acc = jnp.where(lane == j, s, acc)
            res = 1.0 / (1.0 + jnp.exp(-acc))
            out_v[pl.ds(ch * CHUNK + g * LANES, LANES)] = res
        return 0

    lax.fori_loop(0, N_CHUNKS, chunk_body, 0)
    pltpu.sync_copy(out_v, out_hbm.at[pl.ds(base, B_PER_W)])


@jax.jit
def kernel(users, items, user_table, item_table):
    mesh = plsc.VectorSubcoreMesh(core_axis_name="c", subcore_axis_name="s")
    k = pl.kernel(
        _body,
        out_type=jax.ShapeDtypeStruct((BATCH,), jnp.float32),
        mesh=mesh,
        compiler_params=pltpu.CompilerParams(needs_layout_passes=False),
        scratch_types=[
            pltpu.VMEM((CHUNK,), jnp.int32),
            pltpu.VMEM((CHUNK,), jnp.int32),
            pltpu.VMEM((CHUNK, EMBED), jnp.float32),
            pltpu.VMEM((CHUNK, EMBED), jnp.float32),
            pltpu.VMEM((B_PER_W,), jnp.float32),
            pltpu.SemaphoreType.DMA,
        ],
    )
    return k(users, items, user_table, item_table)
